# baseline (speedup 1.0000x reference)
import functools
import math

import jax
import jax.numpy as jnp
import numpy as np
from jax.experimental import pallas as pl
from jax.experimental.pallas import tpu as pltpu

_SIZES = [32, 64, 128, 256, 512]
_RATIOS = [0.5, 1.0, 2.0]

_LEVELS = [(64, 64), (32, 32), (16, 16)]


@functools.lru_cache(maxsize=None)
def _anchors_const(img_h, grids):
    per_all = []
    for grid in grids:
        scale = img_h / grid
        steps = (np.arange(grid, dtype=np.float32)
                 * np.float32(scale)).astype(np.float32)
        x, y = np.meshgrid(steps, steps, indexing='ij')
        for s in _SIZES:
            for r in _RATIOS:
                rs = math.sqrt(r)
                aw = np.full((grid, grid), np.float32(s * rs), dtype=np.float32)
                ah = np.full((grid, grid), np.float32(s / rs), dtype=np.float32)
                a = np.stack((x, y, aw, ah)).transpose(1, 2, 0).reshape(-1, 4)
                per_all.append(a)
    return np.concatenate(per_all, axis=0)


def _rpn_head_kernel(x3, x4, x5, w9, cb, hw_, hb, o3, o4, o5, xs):
    for (h, w), x, o in zip(_LEVELS, (x3, x4, x5), (o3, o4, o5)):
        hw = h * w
        m = w + 1

        xs[:, 0:m] = jnp.zeros((256, m), dtype=jnp.bfloat16)
        xs[:, m + hw:2 * m + hw] = jnp.zeros((256, m), dtype=jnp.bfloat16)
        xs[:, m:m + hw] = x[...].astype(jnp.bfloat16)

        group = []
        for dx in (-1, 0, 1):
            a = None
            for dy in (-1, 0, 1):
                k = (dy + 1) * 3 + (dx + 1)
                off = m + dy * w + dx
                d = jnp.dot(w9[k], xs[:, off:off + hw],
                            preferred_element_type=jnp.float32)
                a = d if a is None else a + d
            group.append(a)

        col = jax.lax.broadcasted_iota(jnp.int32, (1, hw), 1) % w
        acc = (group[1] + cb[...]
               + jnp.where(col != 0, group[0], 0.0)
               + jnp.where(col != w - 1, group[2], 0.0))
        t = jnp.maximum(acc, 0.0).astype(jnp.bfloat16)
        o[...] = jnp.dot(hw_[...], t,
                         preferred_element_type=jnp.float32) + hb[...]


def kernel(images, feat_p3, feat_p4, feat_p5, conv_w, conv_b,
           cls_w, cls_b, bbox_w, bbox_b):
    feats = (feat_p3, feat_p4, feat_p5)
    xs_in = [f.reshape(256, h * w) for f, (h, w) in zip(feats, _LEVELS)]

    w9 = conv_w.transpose(2, 3, 0, 1).reshape(9, 256, 256).astype(jnp.bfloat16)
    cb = conv_b.reshape(256, 1)
    hw_ = jnp.concatenate(
        [cls_w.reshape(15, 256), bbox_w.reshape(60, 256)]).astype(jnp.bfloat16)
    hb = jnp.concatenate([cls_b, bbox_b]).reshape(75, 1)

    out_shapes = tuple(jax.ShapeDtypeStruct((75, h * w), jnp.float32)
                       for h, w in _LEVELS)
    hmax, wmax = _LEVELS[0]

    o3, o4, o5 = pl.pallas_call(
        _rpn_head_kernel,
        out_shape=out_shapes,
        scratch_shapes=[
            pltpu.VMEM((256, hmax * wmax + 2 * wmax + 2), jnp.bfloat16)],
    )(xs_in[0], xs_in[1], xs_in[2], w9, cb, hw_, hb)

    flat = jnp.zeros((1, 403200), dtype=jnp.float32)
    offs, segs = [], []
    off = 0
    for h, w in _LEVELS:
        offs.append(off)
        segs.append(75 * h * w)
        off += segs[-1]
    for i in (2, 1, 0):
        o = (o3, o4, o5)[i]
        flat = jax.lax.dynamic_update_slice(
            flat, o.reshape(1, segs[i]), (0, offs[i]))

    anchors = jnp.asarray(
        _anchors_const(images.shape[-2], tuple(h for h, _ in _LEVELS)))
    return (flat, anchors)


# speedup vs baseline: 1.0584x; 1.0584x over previous
import functools
import math

import jax
import jax.numpy as jnp
import numpy as np
from jax.experimental import pallas as pl
from jax.experimental.pallas import tpu as pltpu

_SIZES = [32, 64, 128, 256, 512]
_RATIOS = [0.5, 1.0, 2.0]

_LEVELS = [(64, 64), (32, 32), (16, 16)]


@functools.lru_cache(maxsize=None)
def _anchors_const(img_h, grids):
    per_all = []
    for grid in grids:
        scale = img_h / grid
        steps = (np.arange(grid, dtype=np.float32)
                 * np.float32(scale)).astype(np.float32)
        x, y = np.meshgrid(steps, steps, indexing='ij')
        for s in _SIZES:
            for r in _RATIOS:
                rs = math.sqrt(r)
                aw = np.full((grid, grid), np.float32(s * rs), dtype=np.float32)
                ah = np.full((grid, grid), np.float32(s / rs), dtype=np.float32)
                a = np.stack((x, y, aw, ah)).transpose(1, 2, 0).reshape(-1, 4)
                per_all.append(a)
    return np.concatenate(per_all, axis=0)


def _rpn_head_kernel(x3, x4, x5, w9, cb, hw_, hb, o3, o4, o5, xs):
    for (h, w), x, o in zip(_LEVELS, (x3, x4, x5), (o3, o4, o5)):
        hw = h * w
        m = w + 1

        xs[:, 0:m] = jnp.zeros((256, m), dtype=jnp.bfloat16)
        xs[:, m + hw:2 * m + hw] = jnp.zeros((256, m), dtype=jnp.bfloat16)
        xs[:, m:m + hw] = x[...]

        group = []
        for dx in (-1, 0, 1):
            a = None
            for dy in (-1, 0, 1):
                k = (dy + 1) * 3 + (dx + 1)
                off = m + dy * w + dx
                d = jnp.dot(w9[k], xs[:, off:off + hw],
                            preferred_element_type=jnp.float32)
                a = d if a is None else a + d
            group.append(a)

        col = jax.lax.broadcasted_iota(jnp.int32, (1, hw), 1) % w
        acc = (group[1] + cb[...]
               + jnp.where(col != 0, group[0], 0.0)
               + jnp.where(col != w - 1, group[2], 0.0))
        t = jnp.maximum(acc, 0.0).astype(jnp.bfloat16)
        o[...] = jnp.dot(hw_[...], t,
                         preferred_element_type=jnp.float32) + hb[...]


def kernel(images, feat_p3, feat_p4, feat_p5, conv_w, conv_b,
           cls_w, cls_b, bbox_w, bbox_b):
    feats = (feat_p3, feat_p4, feat_p5)
    xs_in = [f.reshape(256, h * w).astype(jnp.bfloat16)
             for f, (h, w) in zip(feats, _LEVELS)]

    w9 = conv_w.transpose(2, 3, 0, 1).reshape(9, 256, 256).astype(jnp.bfloat16)
    cb = conv_b.reshape(256, 1)
    hw_ = jnp.concatenate(
        [cls_w.reshape(15, 256), bbox_w.reshape(60, 256)]).astype(jnp.bfloat16)
    hb = jnp.concatenate([cls_b, bbox_b]).reshape(75, 1)

    out_shapes = tuple(jax.ShapeDtypeStruct((75, h * w), jnp.float32)
                       for h, w in _LEVELS)
    hmax, wmax = _LEVELS[0]

    o3, o4, o5 = pl.pallas_call(
        _rpn_head_kernel,
        out_shape=out_shapes,
        scratch_shapes=[
            pltpu.VMEM((256, hmax * wmax + 2 * wmax + 2), jnp.bfloat16)],
    )(xs_in[0], xs_in[1], xs_in[2], w9, cb, hw_, hb)

    flat = jnp.zeros((403200,), dtype=jnp.float32)
    off = 0
    for (h, w), o in zip(_LEVELS, (o3, o4, o5)):
        seg = 75 * h * w
        flat = jax.lax.dynamic_update_slice(flat, o.reshape(seg), (off,))
        off += seg
    flat = flat.reshape(1, 403200)

    anchors = jnp.asarray(
        _anchors_const(images.shape[-2], tuple(h for h, _ in _LEVELS)))
    return (flat, anchors)


# speedup vs baseline: 1.1139x; 1.0524x over previous
import functools
import math

import jax
import jax.numpy as jnp
import numpy as np
from jax.experimental import pallas as pl
from jax.experimental.pallas import tpu as pltpu

_SIZES = [32, 64, 128, 256, 512]
_RATIOS = [0.5, 1.0, 2.0]

_LEVELS = [(64, 64), (32, 32), (16, 16)]


@functools.lru_cache(maxsize=None)
def _anchors_const(img_h, grids):
    per_all = []
    for grid in grids:
        scale = img_h / grid
        steps = (np.arange(grid, dtype=np.float32)
                 * np.float32(scale)).astype(np.float32)
        x, y = np.meshgrid(steps, steps, indexing='ij')
        for s in _SIZES:
            for r in _RATIOS:
                rs = math.sqrt(r)
                aw = np.full((grid, grid), np.float32(s * rs), dtype=np.float32)
                ah = np.full((grid, grid), np.float32(s / rs), dtype=np.float32)
                a = np.stack((x, y, aw, ah)).transpose(1, 2, 0).reshape(-1, 4)
                per_all.append(a)
    return np.concatenate(per_all, axis=0)


def _rpn_head_kernel(xc, w9, cb, hw_, hb, o3, o4, o5, xs):
    xoff = 0
    for (h, w), o in zip(_LEVELS, (o3, o4, o5)):
        hw = h * w
        m = w + 1

        xs[:, 0:m] = jnp.zeros((256, m), dtype=jnp.bfloat16)
        xs[:, m + hw:2 * m + hw] = jnp.zeros((256, m), dtype=jnp.bfloat16)
        xs[:, m:m + hw] = xc[:, xoff:xoff + hw]
        xoff += hw

        group = []
        for dx in (-1, 0, 1):
            a = None
            for dy in (-1, 0, 1):
                k = (dy + 1) * 3 + (dx + 1)
                off = m + dy * w + dx
                d = jnp.dot(w9[k], xs[:, off:off + hw],
                            preferred_element_type=jnp.float32)
                a = d if a is None else a + d
            group.append(a)

        col = jax.lax.broadcasted_iota(jnp.int32, (1, hw), 1) % w
        acc = (group[1] + cb[...]
               + jnp.where(col != 0, group[0], 0.0)
               + jnp.where(col != w - 1, group[2], 0.0))
        t = jnp.maximum(acc, 0.0).astype(jnp.bfloat16)
        o[...] = jnp.dot(hw_[...], t,
                         preferred_element_type=jnp.float32) + hb[...]


def kernel(images, feat_p3, feat_p4, feat_p5, conv_w, conv_b,
           cls_w, cls_b, bbox_w, bbox_b):
    feats = (feat_p3, feat_p4, feat_p5)
    xcat = jnp.concatenate(
        [f.reshape(256, h * w) for f, (h, w) in zip(feats, _LEVELS)],
        axis=1).astype(jnp.bfloat16)

    w9 = conv_w.transpose(2, 3, 0, 1).reshape(9, 256, 256).astype(jnp.bfloat16)
    cb = conv_b.reshape(256, 1)
    hw_ = jnp.concatenate(
        [cls_w.reshape(15, 256), bbox_w.reshape(60, 256)]).astype(jnp.bfloat16)
    hb = jnp.concatenate([cls_b, bbox_b]).reshape(75, 1)

    out_shapes = tuple(jax.ShapeDtypeStruct((75, h * w), jnp.float32)
                       for h, w in _LEVELS)
    hmax, wmax = _LEVELS[0]

    o3, o4, o5 = pl.pallas_call(
        _rpn_head_kernel,
        out_shape=out_shapes,
        scratch_shapes=[
            pltpu.VMEM((256, hmax * wmax + 2 * wmax + 2), jnp.bfloat16)],
    )(xcat, w9, cb, hw_, hb)

    flat = jnp.zeros((1, 403200), dtype=jnp.float32)
    off = 0
    for (h, w), o in zip(_LEVELS, (o3, o4, o5)):
        seg = 75 * h * w
        flat = jax.lax.dynamic_update_slice(flat, o.reshape(1, seg), (0, off))
        off += seg

    anchors = jnp.asarray(
        _anchors_const(images.shape[-2], tuple(h for h, _ in _LEVELS)))
    return (flat, anchors)


# speedup vs baseline: 1.1288x; 1.0134x over previous
import functools
import math

import jax
import jax.numpy as jnp
import numpy as np
from jax.experimental import pallas as pl
from jax.experimental.pallas import tpu as pltpu

_SIZES = [32, 64, 128, 256, 512]
_RATIOS = [0.5, 1.0, 2.0]

_LEVELS = [(64, 64), (32, 32), (16, 16)]


@functools.lru_cache(maxsize=None)
def _anchors_const(img_h, grids):
    per_all = []
    for grid in grids:
        scale = img_h / grid
        steps = (np.arange(grid, dtype=np.float32)
                 * np.float32(scale)).astype(np.float32)
        x, y = np.meshgrid(steps, steps, indexing='ij')
        for s in _SIZES:
            for r in _RATIOS:
                rs = math.sqrt(r)
                aw = np.full((grid, grid), np.float32(s * rs), dtype=np.float32)
                ah = np.full((grid, grid), np.float32(s / rs), dtype=np.float32)
                a = np.stack((x, y, aw, ah)).transpose(1, 2, 0).reshape(-1, 4)
                per_all.append(a)
    return np.concatenate(per_all, axis=0)


def _rpn_head_kernel(x3, x4, x5, w9, cb, hw_, hb, o3, o4, o5, xs):
    for (h, w), x, o in zip(_LEVELS, (x3, x4, x5), (o3, o4, o5)):
        hw = h * w
        m = w + 1

        xs[:, 0:m] = jnp.zeros((256, m), dtype=jnp.bfloat16)
        xs[:, m + hw:2 * m + hw] = jnp.zeros((256, m), dtype=jnp.bfloat16)
        xs[:, m:m + hw] = x[...]

        group = []
        for dx in (-1, 0, 1):
            a = None
            for dy in (-1, 0, 1):
                k = (dy + 1) * 3 + (dx + 1)
                off = m + dy * w + dx
                d = jnp.dot(w9[k], xs[:, off:off + hw],
                            preferred_element_type=jnp.float32)
                a = d if a is None else a + d
            group.append(a)

        col = jax.lax.broadcasted_iota(jnp.int32, (1, hw), 1) % w
        acc = (group[1] + cb[...]
               + jnp.where(col != 0, group[0], 0.0)
               + jnp.where(col != w - 1, group[2], 0.0))
        t = jnp.maximum(acc, 0.0).astype(jnp.bfloat16)
        o[...] = jnp.dot(hw_[...], t,
                         preferred_element_type=jnp.float32) + hb[...]


def kernel(images, feat_p3, feat_p4, feat_p5, conv_w, conv_b,
           cls_w, cls_b, bbox_w, bbox_b):
    feats = (feat_p3, feat_p4, feat_p5)
    xs_in = [f.reshape(256, h * w).astype(jnp.bfloat16)
             for f, (h, w) in zip(feats, _LEVELS)]

    w9 = conv_w.transpose(2, 3, 0, 1).reshape(9, 256, 256).astype(jnp.bfloat16)
    cb = conv_b.reshape(256, 1)
    hw_ = jnp.concatenate(
        [cls_w.reshape(15, 256), bbox_w.reshape(60, 256)]).astype(jnp.bfloat16)
    hb = jnp.concatenate([cls_b, bbox_b]).reshape(75, 1)

    out_shapes = tuple(jax.ShapeDtypeStruct((75, h * w), jnp.float32)
                       for h, w in _LEVELS)
    hmax, wmax = _LEVELS[0]

    o3, o4, o5 = pl.pallas_call(
        _rpn_head_kernel,
        out_shape=out_shapes,
        scratch_shapes=[
            pltpu.VMEM((256, hmax * wmax + 2 * wmax + 2), jnp.bfloat16)],
    )(xs_in[0], xs_in[1], xs_in[2], w9, cb, hw_, hb)

    flat = jnp.zeros((1, 403200), dtype=jnp.float32)
    off = 0
    for (h, w), o in zip(_LEVELS, (o3, o4, o5)):
        seg = 75 * h * w
        flat = jax.lax.dynamic_update_slice(flat, o.reshape(1, seg), (0, off))
        off += seg

    anchors = jnp.asarray(
        _anchors_const(images.shape[-2], tuple(h for h, _ in _LEVELS)))
    return (flat, anchors)
